# Initial kernel scaffold; baseline (speedup 1.0000x reference)
import jax
import jax.numpy as jnp
from jax.experimental import pallas as pl


def kernel(pegs, total_reward, n_pegs, done, actions):
    raise NotImplementedError("write your pallas kernel here")


# speedup vs baseline: 6.3832x; 6.3832x over previous
import functools
import numpy as np
import jax
import jax.numpy as jnp
from jax.experimental import pallas as pl

_N_PEGS = 32
_N_POS = _N_PEGS + 1
_N_ACT = _N_POS * 4


def _build_tables():
    cells = [(i, j) for i in range(7) for j in range(7) if (2 <= i <= 4) or (2 <= j <= 4)]
    center = (3, 3)
    grid = [center] + [c for c in cells if c != center]
    moves = np.array([(-1, 0), (1, 0), (0, -1), (0, 1)], dtype=np.int64)
    grid_arr = np.array(grid, dtype=np.int64)
    pos_to_idx = {tuple(p): k for k, p in enumerate(grid)}
    action_pos_ids = np.repeat(np.arange(_N_POS), 4)
    action_move_ids = np.tile(np.arange(4), _N_POS)
    action_moves = moves[action_move_ids]
    action_positions = grid_arr[action_pos_ids]
    mid_pos = action_positions + action_moves
    tgt_pos = action_positions + 2 * action_moves
    mid_idx = np.zeros(_N_ACT, dtype=np.int64)
    tgt_idx = np.zeros(_N_ACT, dtype=np.int64)
    oob = np.zeros(_N_ACT, dtype=bool)
    for a in range(_N_ACT):
        m = tuple(mid_pos[a]); t = tuple(tgt_pos[a])
        if (m in pos_to_idx) and (t in pos_to_idx):
            mid_idx[a] = pos_to_idx[m]; tgt_idx[a] = pos_to_idx[t]
        else:
            oob[a] = True
            mid_idx[a] = pos_to_idx.get(m, 0)
            tgt_idx[a] = pos_to_idx.get(t, 0)

    p = np.arange(_N_POS)
    set_t = (p[None, :] == tgt_idx[:, None]).astype(np.float32)
    clr_t = (((p[None, :] == action_pos_ids[:, None]) |
              (p[None, :] == mid_idx[:, None])) & (set_t == 0)).astype(np.float32)
    clrset = np.concatenate([clr_t, set_t], axis=1)

    pa = (p[:, None] == action_pos_ids[None, :]).astype(np.float32)
    pm = (p[:, None] == mid_idx[None, :]).astype(np.float32)
    pt = (p[:, None] == tgt_idx[None, :]).astype(np.float32)
    pamt = np.concatenate([pa, pm, pt], axis=1)

    g147 = np.zeros((_N_POS, 147), dtype=np.float32)
    m1 = np.zeros((1, 147), dtype=np.float32)
    m2 = np.zeros((1, 147), dtype=np.float32)
    for i in range(7):
        for j in range(7):
            base = (i * 7 + j) * 3
            if (i, j) in pos_to_idx:
                g147[pos_to_idx[(i, j)], base] = 1.0
            m1[0, base + 1] = 1.0
            m2[0, base + 2] = 1.0

    oob_row = oob.astype(np.int32)[None, :]
    return (jnp.asarray(clrset), jnp.asarray(pamt), jnp.asarray(g147),
            jnp.asarray(m1), jnp.asarray(m2), jnp.asarray(oob_row))


_CLRSET, _PAMT, _G147, _M1, _M2, _OOBROW = _build_tables()


def _step_kernel(pegs_ref, act_ref, npegs_ref, done_ref, trew_ref,
                 clrset_ref, pamt_ref, g147_ref, m1_ref, m2_ref, oob_ref,
                 states_ref, rew_ref, ndone_ref, npegs_out_ref, trew_out_ref):
    pegs = pegs_ref[...]
    a = act_ref[0, 0, :]
    n_pegs = npegs_ref[0, 0, :]
    done = done_ref[0, 0, :]
    trew = trew_ref[0, 0, :]

    b = pegs.shape[0]
    iota_a = jax.lax.broadcasted_iota(jnp.int32, (b, _N_ACT), 1)
    onehot = (a[:, None] == iota_a).astype(jnp.float32)
    cs = jnp.dot(onehot, clrset_ref[...],
                 preferred_element_type=jnp.float32)
    clr = cs[:, :_N_POS]
    st = cs[:, _N_POS:]
    pegs_new = pegs * (1.0 - clr - st) + st

    gath = jnp.dot(pegs_new, pamt_ref[...],
                   preferred_element_type=jnp.float32)
    p_src = gath[:, :_N_ACT]
    p_mid = gath[:, _N_ACT:2 * _N_ACT]
    p_tgt = gath[:, 2 * _N_ACT:]
    oob = oob_ref[0, :][None, :]
    mask = ((p_src != 0.0) & (p_mid > 0.0) & (p_tgt == 0.0)
            & (oob == 0) & (done[:, None] == 0))
    has_feasible = jnp.any(mask, axis=1)

    n_new = n_pegs - 1
    done_win = n_new == 1
    rewards = jnp.where(done_win, 1.0, 1.0 / (_N_PEGS - 1)).astype(jnp.float32)
    new_done = (done_win | (~has_feasible)).astype(jnp.int32)

    nf = n_new.astype(jnp.float32)
    ratio1 = (nf - 1.0) / (_N_PEGS - 1)
    ratio2 = (_N_PEGS - nf) / (_N_PEGS - 1)
    states = (jnp.dot(pegs_new, g147_ref[...],
                      preferred_element_type=jnp.float32)
              + ratio1[:, None] * m1_ref[0, :][None, :]
              + ratio2[:, None] * m2_ref[0, :][None, :])

    states_ref[...] = states
    rew_ref[0, 0, :] = rewards
    ndone_ref[0, 0, :] = new_done
    npegs_out_ref[0, 0, :] = n_new
    trew_out_ref[0, 0, :] = trew + rewards


@jax.jit
def kernel(pegs, total_reward, n_pegs, done, actions):
    n = pegs.shape[0]
    blk = 2048
    grid = n // blk

    act3 = actions.reshape(grid, 1, blk)
    npegs3 = n_pegs.reshape(grid, 1, blk)
    done3 = done.astype(jnp.int32).reshape(grid, 1, blk)
    trew3 = total_reward.reshape(grid, 1, blk)

    row_spec = pl.BlockSpec((1, 1, blk), lambda i: (i, 0, 0))
    full = lambda shape: pl.BlockSpec(shape, lambda i: tuple(0 for _ in shape))

    out_shapes = (
        jax.ShapeDtypeStruct((n, 147), jnp.float32),
        jax.ShapeDtypeStruct((grid, 1, blk), jnp.float32),
        jax.ShapeDtypeStruct((grid, 1, blk), jnp.int32),
        jax.ShapeDtypeStruct((grid, 1, blk), jnp.int32),
        jax.ShapeDtypeStruct((grid, 1, blk), jnp.float32),
    )
    out_specs = (
        pl.BlockSpec((blk, 147), lambda i: (i, 0)),
        row_spec, row_spec, row_spec, row_spec,
    )
    in_specs = [
        pl.BlockSpec((blk, _N_POS), lambda i: (i, 0)),
        row_spec, row_spec, row_spec, row_spec,
        full(_CLRSET.shape), full(_PAMT.shape), full(_G147.shape),
        full(_M1.shape), full(_M2.shape), full(_OOBROW.shape),
    ]

    states_flat, rew, ndone, npo, trew_o = pl.pallas_call(
        _step_kernel,
        grid=(grid,),
        in_specs=in_specs,
        out_specs=out_specs,
        out_shape=out_shapes,
    )(pegs, act3, npegs3, done3, trew3,
      _CLRSET, _PAMT, _G147, _M1, _M2, _OOBROW)

    states = states_flat.reshape(n, 7, 7, 3)
    return (rew.reshape(n), states, ndone.reshape(n).astype(jnp.bool_),
            npo.reshape(n), trew_o.reshape(n))
